# baseline (speedup 1.0000x reference)
import functools

import jax
import jax.numpy as jnp
from jax import lax
from jax.experimental import pallas as pl
from jax.experimental.pallas import tpu as pltpu
from jax.experimental.pallas import tpu_sc as plsc

N_COL = 3200000
O_COL = N_COL // 2
NUM_CORES = 2
NUM_SUBCORES = 16
NW = NUM_CORES * NUM_SUBCORES
C_IN = 6400
C_OUT = C_IN // 2
N_CHUNK = N_COL // C_IN
MAX_J = -(-N_CHUNK // NW)
FULL_W = N_CHUNK - (MAX_J - 1) * NW

_mesh = plsc.VectorSubcoreMesh(core_axis_name="c", subcore_axis_name="s")


@functools.partial(
    pl.kernel,
    mesh=_mesh,
    out_type=jax.ShapeDtypeStruct((2, O_COL), jnp.int32),
    scratch_types=[
        pltpu.VMEM((2, C_IN), jnp.int32),
        pltpu.VMEM((2, C_IN), jnp.int32),
        pltpu.VMEM((2, C_OUT), jnp.int32),
        pltpu.VMEM((2, C_OUT), jnp.int32),
        pltpu.SemaphoreType.DMA,
        pltpu.SemaphoreType.DMA,
        pltpu.SemaphoreType.DMA,
        pltpu.SemaphoreType.DMA,
    ],
    compiler_params=pltpu.CompilerParams(needs_layout_passes=False),
)
def _dilate_sc(in_hbm, out_hbm, in0, in1, out0, out1, si0, si1, so0, so1):
    wid = lax.axis_index("s") * NUM_CORES + lax.axis_index("c")
    iota2 = lax.iota(jnp.int32, 16) * 2
    row_vecs = (jnp.zeros((16,), jnp.int32), jnp.ones((16,), jnp.int32))

    ins, outs = (in0, in1), (out0, out1)
    sis, sos = (si0, si1), (so0, so1)

    def in_copy(j, b):
        k = wid + NW * j
        return pltpu.make_async_copy(
            in_hbm.at[:, pl.ds(k * C_IN, C_IN)], ins[b], sis[b])

    def out_copy(j, b):
        k = wid + NW * j
        return pltpu.make_async_copy(
            outs[b], out_hbm.at[:, pl.ds(k * C_OUT, C_OUT)], sos[b])

    def do_chunk(j, b):
        in_copy(j, b).wait()
        if j >= 2:
            out_copy(j - 2, b).wait()
        src, dst = ins[b], outs[b]
        for r in (0, 1):
            rv = row_vecs[r]

            @plsc.parallel_loop(0, C_OUT // 16, 1, unroll=16)
            def _(i, src=src, dst=dst, r=r, rv=rv):
                idx = iota2 + i * 32
                dst[r, pl.ds(i * 16, 16)] = plsc.load_gather(src, [rv, idx])

        out_copy(j, b).start()

    in_copy(0, 0).start()
    for j in range(MAX_J):
        b = j & 1
        if j + 1 < MAX_J - 1:
            in_copy(j + 1, 1 - b).start()
        elif j + 1 == MAX_J - 1:
            @pl.when(wid < FULL_W)
            def _():
                in_copy(MAX_J - 1, 1 - b).start()
        if j < MAX_J - 1:
            do_chunk(j, b)
        else:
            @pl.when(wid < FULL_W)
            def _():
                do_chunk(j, b)

    out_copy(MAX_J - 2, 0).wait()
    out_copy(MAX_J - 3, 1).wait()


def kernel(edge_index):
    return _dilate_sc(edge_index)


# speedup vs baseline: 1.2273x; 1.2273x over previous
import functools

import jax
import jax.numpy as jnp
from jax import lax
from jax.experimental import pallas as pl
from jax.experimental.pallas import tpu as pltpu
from jax.experimental.pallas import tpu_sc as plsc

N_COL = 3200000
O_COL = N_COL // 2
NUM_CORES = 2
NUM_SUBCORES = 16
NW = NUM_CORES * NUM_SUBCORES
C_IN = 25600
C_OUT = C_IN // 2
N_CHUNK = N_COL // C_IN
MAX_J = -(-N_CHUNK // NW)
FULL_W = N_CHUNK - (MAX_J - 1) * NW

_mesh = plsc.VectorSubcoreMesh(core_axis_name="c", subcore_axis_name="s")


@functools.partial(
    pl.kernel,
    mesh=_mesh,
    out_type=jax.ShapeDtypeStruct((2, O_COL), jnp.int32),
    scratch_types=[
        pltpu.VMEM((2, C_IN), jnp.int32),
        pltpu.VMEM((2, C_IN), jnp.int32),
        pltpu.VMEM((2, C_OUT), jnp.int32),
        pltpu.SemaphoreType.DMA,
        pltpu.SemaphoreType.DMA,
        pltpu.SemaphoreType.DMA,
    ],
    compiler_params=pltpu.CompilerParams(needs_layout_passes=False),
)
def _dilate_sc(in_hbm, out_hbm, in0, in1, out_v, si0, si1, so):
    wid = lax.axis_index("s") * NUM_CORES + lax.axis_index("c")
    iota2 = lax.iota(jnp.int32, 16) * 2
    row_vecs = (jnp.zeros((16,), jnp.int32), jnp.ones((16,), jnp.int32))

    ins, sis = (in0, in1), (si0, si1)

    def in_copy(j, b):
        k = wid + NW * j
        return pltpu.make_async_copy(
            in_hbm.at[:, pl.ds(k * C_IN, C_IN)], ins[b], sis[b])

    def out_copy(j):
        k = wid + NW * j
        return pltpu.make_async_copy(
            out_v, out_hbm.at[:, pl.ds(k * C_OUT, C_OUT)], so)

    def do_chunk(j, b):
        in_copy(j, b).wait()
        if j >= 1:
            out_copy(j - 1).wait()
        src = ins[b]
        for r in (0, 1):
            rv = row_vecs[r]

            @plsc.parallel_loop(0, C_OUT // 16, 1, unroll=16)
            def _(i, src=src, r=r, rv=rv):
                idx = iota2 + i * 32
                out_v[r, pl.ds(i * 16, 16)] = plsc.load_gather(src, [rv, idx])

        out_copy(j).start()

    in_copy(0, 0).start()
    for j in range(MAX_J):
        b = j & 1
        if j + 1 < MAX_J - 1:
            in_copy(j + 1, 1 - b).start()
        elif j + 1 == MAX_J - 1:
            @pl.when(wid < FULL_W)
            def _():
                in_copy(MAX_J - 1, 1 - b).start()
        if j < MAX_J - 1:
            do_chunk(j, b)
        else:
            @pl.when(wid < FULL_W)
            def _():
                do_chunk(j, b)

    out_copy(MAX_J - 2).wait()


def kernel(edge_index):
    return _dilate_sc(edge_index)


# speedup vs baseline: 1.2288x; 1.0012x over previous
import functools

import jax
import jax.numpy as jnp
from jax import lax
from jax.experimental import pallas as pl
from jax.experimental.pallas import tpu as pltpu
from jax.experimental.pallas import tpu_sc as plsc

N_COL = 3200000
O_COL = N_COL // 2
NUM_CORES = 2
NUM_SUBCORES = 16
NW = NUM_CORES * NUM_SUBCORES
C_IN = 25600
C_OUT = C_IN // 2
N_CHUNK = N_COL // C_IN
MAX_J = -(-N_CHUNK // NW)
FULL_W = N_CHUNK - (MAX_J - 1) * NW

_mesh = plsc.VectorSubcoreMesh(core_axis_name="c", subcore_axis_name="s")


@functools.partial(
    pl.kernel,
    mesh=_mesh,
    out_type=jax.ShapeDtypeStruct((2, O_COL), jnp.int32),
    scratch_types=[
        pltpu.VMEM((2, C_IN), jnp.int32),
        pltpu.VMEM((2, C_IN), jnp.int32),
        pltpu.VMEM((2, C_OUT), jnp.int32),
        pltpu.SemaphoreType.DMA,
        pltpu.SemaphoreType.DMA,
        pltpu.SemaphoreType.DMA,
    ],
    compiler_params=pltpu.CompilerParams(
        needs_layout_passes=False,
        disable_bounds_checks=True,
        disable_semaphore_checks=True,
    ),
)
def _dilate_sc(in_hbm, out_hbm, in0, in1, out_v, si0, si1, so):
    wid = lax.axis_index("s") * NUM_CORES + lax.axis_index("c")
    iota2 = lax.iota(jnp.int32, 16) * 2
    row_vecs = (jnp.zeros((16,), jnp.int32), jnp.ones((16,), jnp.int32))

    ins, sis = (in0, in1), (si0, si1)

    def in_copy(j, b):
        k = wid + NW * j
        return pltpu.make_async_copy(
            in_hbm.at[:, pl.ds(k * C_IN, C_IN)], ins[b], sis[b])

    def out_copy(j):
        k = wid + NW * j
        return pltpu.make_async_copy(
            out_v, out_hbm.at[:, pl.ds(k * C_OUT, C_OUT)], so)

    def do_chunk(j, b):
        in_copy(j, b).wait()
        if j >= 1:
            out_copy(j - 1).wait()
        src = ins[b]
        for r in (0, 1):
            rv = row_vecs[r]

            @plsc.parallel_loop(0, C_OUT // 16, 1, unroll=16)
            def _(i, src=src, r=r, rv=rv):
                idx = iota2 + i * 32
                out_v[r, pl.ds(i * 16, 16)] = plsc.load_gather(src, [rv, idx])

        out_copy(j).start()

    in_copy(0, 0).start()
    for j in range(MAX_J):
        b = j & 1
        if j + 1 < MAX_J - 1:
            in_copy(j + 1, 1 - b).start()
        elif j + 1 == MAX_J - 1:
            @pl.when(wid < FULL_W)
            def _():
                in_copy(MAX_J - 1, 1 - b).start()
        if j < MAX_J - 1:
            do_chunk(j, b)
        else:
            @pl.when(wid < FULL_W)
            def _():
                do_chunk(j, b)

    out_copy(MAX_J - 2).wait()


def kernel(edge_index):
    return _dilate_sc(edge_index)


# speedup vs baseline: 1.2324x; 1.0029x over previous
import functools

import jax
import jax.numpy as jnp
from jax import lax
from jax.experimental import pallas as pl
from jax.experimental.pallas import tpu as pltpu
from jax.experimental.pallas import tpu_sc as plsc

N_COL = 3200000
O_COL = N_COL // 2
NUM_CORES = 2
NUM_SUBCORES = 16
NW = NUM_CORES * NUM_SUBCORES
C_IN = 25600
C_OUT = C_IN // 2
N_CHUNK = N_COL // C_IN
MAX_J = -(-N_CHUNK // NW)
FULL_W = N_CHUNK - (MAX_J - 1) * NW

_mesh = plsc.VectorSubcoreMesh(core_axis_name="c", subcore_axis_name="s")


@functools.partial(
    pl.kernel,
    mesh=_mesh,
    out_type=jax.ShapeDtypeStruct((2, O_COL), jnp.int32),
    scratch_types=[
        pltpu.VMEM((2, C_IN), jnp.int32),
        pltpu.VMEM((2, C_IN), jnp.int32),
        pltpu.VMEM((2, C_OUT), jnp.int32),
        pltpu.SemaphoreType.DMA,
        pltpu.SemaphoreType.DMA,
        pltpu.SemaphoreType.DMA,
    ],
    compiler_params=pltpu.CompilerParams(
        needs_layout_passes=False,
        disable_bounds_checks=True,
        disable_semaphore_checks=True,
        skip_device_barrier=True,
    ),
)
def _dilate_sc(in_hbm, out_hbm, in0, in1, out_v, si0, si1, so):
    wid = lax.axis_index("s") * NUM_CORES + lax.axis_index("c")
    iota2 = lax.iota(jnp.int32, 16) * 2
    row_vecs = (jnp.zeros((16,), jnp.int32), jnp.ones((16,), jnp.int32))

    ins, sis = (in0, in1), (si0, si1)

    def in_copy(j, b):
        k = wid + NW * j
        return pltpu.make_async_copy(
            in_hbm.at[:, pl.ds(k * C_IN, C_IN)], ins[b], sis[b])

    def out_copy(j):
        k = wid + NW * j
        return pltpu.make_async_copy(
            out_v, out_hbm.at[:, pl.ds(k * C_OUT, C_OUT)], so)

    def do_chunk(j, b):
        in_copy(j, b).wait()
        if j >= 1:
            out_copy(j - 1).wait()
        src = ins[b]
        for r in (0, 1):
            rv = row_vecs[r]

            @plsc.parallel_loop(0, C_OUT // 16, 1, unroll=16)
            def _(i, src=src, r=r, rv=rv):
                idx = iota2 + i * 32
                out_v[r, pl.ds(i * 16, 16)] = plsc.load_gather(src, [rv, idx])

        out_copy(j).start()

    in_copy(0, 0).start()
    for j in range(MAX_J):
        b = j & 1
        if j + 1 < MAX_J - 1:
            in_copy(j + 1, 1 - b).start()
        elif j + 1 == MAX_J - 1:
            @pl.when(wid < FULL_W)
            def _():
                in_copy(MAX_J - 1, 1 - b).start()
        if j < MAX_J - 1:
            do_chunk(j, b)
        else:
            @pl.when(wid < FULL_W)
            def _():
                do_chunk(j, b)

    out_copy(MAX_J - 2).wait()


def kernel(edge_index):
    return _dilate_sc(edge_index)
